# baseline (speedup 1.0000x reference)
import functools

import jax
import jax.numpy as jnp
from jax import lax
from jax.experimental import pallas as pl
from jax.experimental.pallas import tpu as pltpu
from jax.experimental.pallas import tpu_sc as plsc

NC = 2
NS = 16
NL = 16
NW = NC * NS

B = 16384
FORMULA_LEN = 3
KCH = (B // NW) // 128


def _body(x_hbm, scal_hbm, mat_hbm, p0_hbm, p1_hbm, p2_hbm, out_hbm,
          x_v, i0_v, i1_v, i2_v, c0_v, c1_v, c2_v, a0_v, a1_v, a2_v,
          scal_v, out_v, sem):
    wid = lax.axis_index("s") * NC + lax.axis_index("c")

    pltpu.sync_copy(x_hbm.at[wid], x_v)
    pltpu.sync_copy(scal_hbm, scal_v)

    for j in range(KCH):
        for k in range(128 // NL):
            xv = x_v[j, pl.ds(k * NL, NL)]
            x3 = xv * 3
            i0_v[j, pl.ds(k * NL, NL)] = x3
            i1_v[j, pl.ds(k * NL, NL)] = x3 + 1
            i2_v[j, pl.ds(k * NL, NL)] = x3 + 2

    copies = []
    for idx_v, col_v in ((i0_v, c0_v), (i1_v, c1_v), (i2_v, c2_v)):
        for j in range(KCH):
            copies.append(
                pltpu.make_async_copy(mat_hbm.at[idx_v.at[j]], col_v.at[j], sem))
    for c in copies:
        c.start()
    for c in copies:
        c.wait()

    copies = []
    for p_hbm, col_v, act_v in ((p0_hbm, c0_v, a0_v),
                                (p1_hbm, c1_v, a1_v),
                                (p2_hbm, c2_v, a2_v)):
        for j in range(KCH):
            copies.append(
                pltpu.make_async_copy(p_hbm.at[col_v.at[j]], act_v.at[j], sem))
    for c in copies:
        c.start()
    for c in copies:
        c.wait()

    w0 = scal_v[0, :]
    w1 = scal_v[1, :]
    w2 = scal_v[2, :]
    beta = scal_v[3, :]
    one = jnp.ones((NL,), jnp.float32)
    zero = jnp.zeros((NL,), jnp.float32)
    for j in range(KCH):
        for k in range(128 // NL):
            sl = pl.ds(k * NL, NL)
            a0 = a0_v[j, sl]
            a1 = a1_v[j, sl]
            a2 = a2_v[j, sl]
            s = w0 * (one - a0) + w1 * (one - a1) + w2 * (one - a2)
            out_v[j, sl] = jnp.minimum(jnp.maximum(beta - s, zero), one)

    pltpu.sync_copy(out_v, out_hbm.at[wid])


@jax.jit
def _run(x_tiles, scal, mat_flat, p0_flat, p1_flat, p2_flat):
    mesh = plsc.VectorSubcoreMesh(core_axis_name="c", subcore_axis_name="s",
                                  num_cores=NC, num_subcores=NS)
    f = pl.kernel(
        _body,
        out_type=jax.ShapeDtypeStruct((NW, KCH, 128), jnp.float32),
        mesh=mesh,
        scratch_types=[
            pltpu.VMEM((KCH, 128), jnp.int32),
            pltpu.VMEM((KCH, 128), jnp.int32),
            pltpu.VMEM((KCH, 128), jnp.int32),
            pltpu.VMEM((KCH, 128), jnp.int32),
            pltpu.VMEM((KCH, 128), jnp.int32),
            pltpu.VMEM((KCH, 128), jnp.int32),
            pltpu.VMEM((KCH, 128), jnp.int32),
            pltpu.VMEM((KCH, 128), jnp.float32),
            pltpu.VMEM((KCH, 128), jnp.float32),
            pltpu.VMEM((KCH, 128), jnp.float32),
            pltpu.VMEM((4, NL), jnp.float32),
            pltpu.VMEM((KCH, 128), jnp.float32),
            pltpu.SemaphoreType.DMA,
        ],
        name="meta_rule_sc",
    )
    return f(x_tiles, scal, mat_flat, p0_flat, p1_flat, p2_flat)


def kernel(x, mat, pred0, pred1, pred2, and_w, and_beta):
    x_tiles = x.astype(jnp.int32).reshape(NW, KCH, 128)
    mat_flat = mat.astype(jnp.int32).reshape(-1)
    p0 = pred0.reshape(-1)
    p1 = pred1.reshape(-1)
    p2 = pred2.reshape(-1)
    scal = jnp.concatenate(
        [jnp.broadcast_to(and_w[i], (1, NL)) for i in range(FORMULA_LEN)]
        + [jnp.broadcast_to(and_beta[0], (1, NL))], axis=0).astype(jnp.float32)
    out = _run(x_tiles, scal, mat_flat, p0, p1, p2)
    ret = out.reshape(B, 1)
    slacks = jnp.zeros((), dtype=jnp.float32)
    return (ret, slacks)


# speedup vs baseline: 15.5628x; 15.5628x over previous
import jax
import jax.numpy as jnp
from jax import lax
from jax.experimental import pallas as pl
from jax.experimental.pallas import tpu as pltpu
from jax.experimental.pallas import tpu_sc as plsc

NC = 2
NS = 16
NL = 16
NW = NC * NS

B = 16384
FORMULA_LEN = 3
KCH = (B // NW) // 128


def _body(x_hbm, scal_hbm, c0_hbm, c1_hbm, c2_hbm, p0_hbm, p1_hbm, p2_hbm,
          out_hbm, x_v, i0_v, i1_v, i2_v, a0_v, a1_v, a2_v, scal_v, out_v,
          sem):
    wid = lax.axis_index("s") * NC + lax.axis_index("c")

    pltpu.sync_copy(x_hbm.at[wid], x_v)
    pltpu.sync_copy(scal_hbm, scal_v)

    copies = []
    for c_hbm, col_v in ((c0_hbm, i0_v), (c1_hbm, i1_v), (c2_hbm, i2_v)):
        for j in range(KCH):
            copies.append(
                pltpu.make_async_copy(c_hbm.at[x_v.at[j]], col_v.at[j], sem))
    for c in copies:
        c.start()
    for c in copies:
        c.wait()

    copies = []
    for p_hbm, col_v, act_v in ((p0_hbm, i0_v, a0_v),
                                (p1_hbm, i1_v, a1_v),
                                (p2_hbm, i2_v, a2_v)):
        for j in range(KCH):
            copies.append(
                pltpu.make_async_copy(p_hbm.at[col_v.at[j]], act_v.at[j], sem))
    for c in copies:
        c.start()
    for c in copies:
        c.wait()

    w0 = scal_v[0, :]
    w1 = scal_v[1, :]
    w2 = scal_v[2, :]
    beta = scal_v[3, :]
    one = jnp.ones((NL,), jnp.float32)
    zero = jnp.zeros((NL,), jnp.float32)
    for j in range(KCH):
        for t in range(128 // NL):
            sl = pl.ds(t * NL, NL)
            a0 = a0_v[j, sl]
            a1 = a1_v[j, sl]
            a2 = a2_v[j, sl]
            s = w0 * (one - a0) + w1 * (one - a1) + w2 * (one - a2)
            out_v[j, sl] = jnp.minimum(jnp.maximum(beta - s, zero), one)

    pltpu.sync_copy(out_v, out_hbm.at[wid])


@jax.jit
def _run(x_tiles, scal, c0, c1, c2, p0, p1, p2):
    mesh = plsc.VectorSubcoreMesh(core_axis_name="c", subcore_axis_name="s",
                                  num_cores=NC, num_subcores=NS)
    f = pl.kernel(
        _body,
        out_type=jax.ShapeDtypeStruct((NW, KCH, 128), jnp.float32),
        mesh=mesh,
        scratch_types=[
            pltpu.VMEM((KCH, 128), jnp.int32),
            pltpu.VMEM((KCH, 128), jnp.int32),
            pltpu.VMEM((KCH, 128), jnp.int32),
            pltpu.VMEM((KCH, 128), jnp.int32),
            pltpu.VMEM((KCH, 128), jnp.float32),
            pltpu.VMEM((KCH, 128), jnp.float32),
            pltpu.VMEM((KCH, 128), jnp.float32),
            pltpu.VMEM((4, NL), jnp.float32),
            pltpu.VMEM((KCH, 128), jnp.float32),
            pltpu.SemaphoreType.DMA,
        ],
        name="meta_rule_sc",
    )
    return f(x_tiles, scal, c0, c1, c2, p0, p1, p2)


def kernel(x, mat, pred0, pred1, pred2, and_w, and_beta):
    x_tiles = x.astype(jnp.int32).reshape(NW, KCH, 128)
    mat32 = mat.astype(jnp.int32)
    c0 = mat32[:, 0]
    c1 = mat32[:, 1]
    c2 = mat32[:, 2]
    p0 = pred0.reshape(-1)
    p1 = pred1.reshape(-1)
    p2 = pred2.reshape(-1)
    scal = jnp.concatenate(
        [jnp.broadcast_to(and_w[i], (1, NL)) for i in range(FORMULA_LEN)]
        + [jnp.broadcast_to(and_beta[0], (1, NL))], axis=0).astype(jnp.float32)
    out = _run(x_tiles, scal, c0, c1, c2, p0, p1, p2)
    ret = out.reshape(B, 1)
    slacks = jnp.zeros((), dtype=jnp.float32)
    return (ret, slacks)


# speedup vs baseline: 18.0426x; 1.1593x over previous
import jax
import jax.numpy as jnp
from jax import lax
from jax.experimental import pallas as pl
from jax.experimental.pallas import tpu as pltpu
from jax.experimental.pallas import tpu_sc as plsc

NC = 2
NS = 16
NL = 16
NW = NC * NS

B = 16384
FORMULA_LEN = 3
N_PRED = 1000000
KCH = (B // NW) // 128


def _body(x_hbm, scal_hbm, mat_hbm, p0_hbm, p1_hbm, p2_hbm,
          out_hbm, x_v, x1_v, x2_v, c0_v, c1_v, c2_v, a0_v, a1_v, a2_v,
          scal_v, out_v, sem):
    wid = lax.axis_index("s") * NC + lax.axis_index("c")

    pltpu.sync_copy(x_hbm.at[wid], x_v)
    pltpu.sync_copy(scal_hbm, scal_v)

    for j in range(KCH):
        for t in range(128 // NL):
            sl = pl.ds(t * NL, NL)
            xv = x_v[j, sl]
            x1_v[j, sl] = xv + N_PRED
            x2_v[j, sl] = xv + 2 * N_PRED

    copies = []
    for idx_v, col_v in ((x_v, c0_v), (x1_v, c1_v), (x2_v, c2_v)):
        for j in range(KCH):
            copies.append(
                pltpu.make_async_copy(mat_hbm.at[idx_v.at[j]], col_v.at[j],
                                      sem))
    for c in copies:
        c.start()
    for c in copies:
        c.wait()

    copies = []
    for p_hbm, col_v, act_v in ((p0_hbm, c0_v, a0_v),
                                (p1_hbm, c1_v, a1_v),
                                (p2_hbm, c2_v, a2_v)):
        for j in range(KCH):
            copies.append(
                pltpu.make_async_copy(p_hbm.at[0].at[col_v.at[j]],
                                      act_v.at[j], sem))
    for c in copies:
        c.start()
    for c in copies:
        c.wait()

    w0 = scal_v[0, :]
    w1 = scal_v[1, :]
    w2 = scal_v[2, :]
    beta = scal_v[3, :]
    one = jnp.ones((NL,), jnp.float32)
    zero = jnp.zeros((NL,), jnp.float32)
    for j in range(KCH):
        for t in range(128 // NL):
            sl = pl.ds(t * NL, NL)
            a0 = a0_v[j, sl]
            a1 = a1_v[j, sl]
            a2 = a2_v[j, sl]
            s = w0 * (one - a0) + w1 * (one - a1) + w2 * (one - a2)
            out_v[j, sl] = jnp.minimum(jnp.maximum(beta - s, zero), one)

    pltpu.sync_copy(out_v, out_hbm.at[wid])


@jax.jit
def _run(x_tiles, scal, matflat, p0, p1, p2):
    mesh = plsc.VectorSubcoreMesh(core_axis_name="c", subcore_axis_name="s",
                                  num_cores=NC, num_subcores=NS)
    f = pl.kernel(
        _body,
        out_type=jax.ShapeDtypeStruct((NW, KCH, 128), jnp.float32),
        mesh=mesh,
        compiler_params=pltpu.CompilerParams(use_tc_tiling_on_sc=False),
        scratch_types=[
            pltpu.VMEM((KCH, 128), jnp.int32),
            pltpu.VMEM((KCH, 128), jnp.int32),
            pltpu.VMEM((KCH, 128), jnp.int32),
            pltpu.VMEM((KCH, 128), jnp.int32),
            pltpu.VMEM((KCH, 128), jnp.int32),
            pltpu.VMEM((KCH, 128), jnp.int32),
            pltpu.VMEM((KCH, 128), jnp.float32),
            pltpu.VMEM((KCH, 128), jnp.float32),
            pltpu.VMEM((KCH, 128), jnp.float32),
            pltpu.VMEM((4, NL), jnp.float32),
            pltpu.VMEM((KCH, 128), jnp.float32),
            pltpu.SemaphoreType.DMA,
        ],
        name="meta_rule_sc",
    )
    return f(x_tiles, scal, matflat, p0, p1, p2)


def kernel(x, mat, pred0, pred1, pred2, and_w, and_beta):
    x_tiles = x.astype(jnp.int32).reshape(NW, KCH, 128)
    matflat = mat.astype(jnp.int32).T.reshape(-1)
    p0 = pred0.reshape(1, -1)
    p1 = pred1.reshape(1, -1)
    p2 = pred2.reshape(1, -1)
    scal = jnp.concatenate(
        [jnp.broadcast_to(and_w[i], (1, NL)) for i in range(FORMULA_LEN)]
        + [jnp.broadcast_to(and_beta[0], (1, NL))], axis=0).astype(jnp.float32)
    out = _run(x_tiles, scal, matflat, p0, p1, p2)
    ret = out.reshape(B, 1)
    slacks = jnp.zeros((), dtype=jnp.float32)
    return (ret, slacks)


# speedup vs baseline: 50.5151x; 2.7998x over previous
import jax
import jax.numpy as jnp
from jax import lax
from jax.experimental import pallas as pl
from jax.experimental.pallas import tpu as pltpu
from jax.experimental.pallas import tpu_sc as plsc

NC = 2
NS = 16
NL = 16
NW = NC * NS

B = 16384
FORMULA_LEN = 3
N_PRED = 1000000
KCH = (B // NW) // 128


def _body(x_hbm, scal_hbm, mat_hbm, p0_hbm, p1_hbm, p2_hbm,
          out_hbm, x_v, x1_v, x2_v, c0_v, c1_v, c2_v, a0_v, a1_v, a2_v,
          scal_v, out_v, sem):
    wid = lax.axis_index("s") * NC + lax.axis_index("c")

    pltpu.sync_copy(x_hbm.at[wid], x_v)
    pltpu.sync_copy(scal_hbm, scal_v)

    for j in range(KCH):
        for t in range(128 // NL):
            sl = pl.ds(t * NL, NL)
            xv = x_v[j, sl]
            x1_v[j, sl] = xv + N_PRED
            x2_v[j, sl] = xv + 2 * N_PRED

    copies = []
    for idx_v, col_v in ((x_v, c0_v), (x1_v, c1_v), (x2_v, c2_v)):
        for j in range(KCH):
            copies.append(
                pltpu.make_async_copy(mat_hbm.at[idx_v.at[j]], col_v.at[j],
                                      sem))
    for c in copies:
        c.start()
    for c in copies:
        c.wait()

    copies = []
    for p_hbm, col_v, act_v in ((p0_hbm, c0_v, a0_v),
                                (p1_hbm, c1_v, a1_v),
                                (p2_hbm, c2_v, a2_v)):
        for j in range(KCH):
            copies.append(
                pltpu.make_async_copy(p_hbm.at[0].at[col_v.at[j]],
                                      act_v.at[j], sem))
    for c in copies:
        c.start()
    for c in copies:
        c.wait()

    w0 = scal_v[0, :]
    w1 = scal_v[1, :]
    w2 = scal_v[2, :]
    beta = scal_v[3, :]
    one = jnp.ones((NL,), jnp.float32)
    zero = jnp.zeros((NL,), jnp.float32)
    for j in range(KCH):
        for t in range(128 // NL):
            sl = pl.ds(t * NL, NL)
            a0 = a0_v[j, sl]
            a1 = a1_v[j, sl]
            a2 = a2_v[j, sl]
            s = w0 * (one - a0) + w1 * (one - a1) + w2 * (one - a2)
            out_v[j, sl] = jnp.minimum(jnp.maximum(beta - s, zero), one)

    pltpu.sync_copy(out_v, out_hbm.at[wid])


@jax.jit
def _run(x_tiles, scal, matflat, p0, p1, p2):
    mesh = plsc.VectorSubcoreMesh(core_axis_name="c", subcore_axis_name="s",
                                  num_cores=NC, num_subcores=NS)
    f = pl.kernel(
        _body,
        out_type=jax.ShapeDtypeStruct((NW, KCH, 128), jnp.float32),
        mesh=mesh,
        scratch_types=[
            pltpu.VMEM((KCH, 128), jnp.int32),
            pltpu.VMEM((KCH, 128), jnp.int32),
            pltpu.VMEM((KCH, 128), jnp.int32),
            pltpu.VMEM((KCH, 128), jnp.int32),
            pltpu.VMEM((KCH, 128), jnp.int32),
            pltpu.VMEM((KCH, 128), jnp.int32),
            pltpu.VMEM((KCH, 128), jnp.float32),
            pltpu.VMEM((KCH, 128), jnp.float32),
            pltpu.VMEM((KCH, 128), jnp.float32),
            pltpu.VMEM((4, NL), jnp.float32),
            pltpu.VMEM((KCH, 128), jnp.float32),
            pltpu.SemaphoreType.DMA,
        ],
        name="meta_rule_sc",
    )
    return f(x_tiles, scal, matflat, p0, p1, p2)


def kernel(x, mat, pred0, pred1, pred2, and_w, and_beta):
    x_tiles = x.astype(jnp.int32).reshape(NW, KCH, 128)
    matflat = mat.astype(jnp.int32).T.reshape(-1)
    p0 = pred0.reshape(1, -1)
    p1 = pred1.reshape(1, -1)
    p2 = pred2.reshape(1, -1)
    scal = jnp.concatenate(
        [jnp.broadcast_to(and_w[i], (1, NL)) for i in range(FORMULA_LEN)]
        + [jnp.broadcast_to(and_beta[0], (1, NL))], axis=0).astype(jnp.float32)
    out = _run(x_tiles, scal, matflat, p0, p1, p2)
    ret = out.reshape(B, 1)
    slacks = jnp.zeros((), dtype=jnp.float32)
    return (ret, slacks)


# speedup vs baseline: 50.6261x; 1.0022x over previous
import jax
import jax.numpy as jnp
from jax import lax
from jax.experimental import pallas as pl
from jax.experimental.pallas import tpu as pltpu
from jax.experimental.pallas import tpu_sc as plsc

NC = 2
NS = 16
NL = 16
NW = NC * NS

B = 16384
FORMULA_LEN = 3
N_PRED = 1000000
KCH = (B // NW) // 128


def _body(x_hbm, scal_hbm, mat_hbm, p0_hbm, p1_hbm, p2_hbm,
          out_hbm, x_v, c0_v, c1_v, c2_v, a0_v, a1_v, a2_v,
          scal_v, out_v, sem):
    wid = lax.axis_index("s") * NC + lax.axis_index("c")

    pltpu.sync_copy(x_hbm.at[wid], x_v)
    pltpu.sync_copy(scal_hbm, scal_v)

    copies = []
    for i, col_v in ((0, c0_v), (1, c1_v), (2, c2_v)):
        view = mat_hbm.at[pl.ds(i * N_PRED, N_PRED)]
        for j in range(KCH):
            copies.append(
                pltpu.make_async_copy(view.at[x_v.at[j]], col_v.at[j], sem))
    for c in copies:
        c.start()
    for c in copies:
        c.wait()

    copies = []
    for p_hbm, col_v, act_v in ((p0_hbm, c0_v, a0_v),
                                (p1_hbm, c1_v, a1_v),
                                (p2_hbm, c2_v, a2_v)):
        for j in range(KCH):
            copies.append(
                pltpu.make_async_copy(p_hbm.at[0].at[col_v.at[j]],
                                      act_v.at[j], sem))
    for c in copies:
        c.start()
    for c in copies:
        c.wait()

    w0 = scal_v[0, :]
    w1 = scal_v[1, :]
    w2 = scal_v[2, :]
    beta = scal_v[3, :]
    one = jnp.ones((NL,), jnp.float32)
    zero = jnp.zeros((NL,), jnp.float32)
    for j in range(KCH):
        for t in range(128 // NL):
            sl = pl.ds(t * NL, NL)
            a0 = a0_v[j, sl]
            a1 = a1_v[j, sl]
            a2 = a2_v[j, sl]
            s = w0 * (one - a0) + w1 * (one - a1) + w2 * (one - a2)
            out_v[j, sl] = jnp.minimum(jnp.maximum(beta - s, zero), one)

    pltpu.sync_copy(out_v, out_hbm.at[wid])


@jax.jit
def _run(x_tiles, scal, matflat, p0, p1, p2):
    mesh = plsc.VectorSubcoreMesh(core_axis_name="c", subcore_axis_name="s",
                                  num_cores=NC, num_subcores=NS)
    f = pl.kernel(
        _body,
        out_type=jax.ShapeDtypeStruct((NW, KCH, 128), jnp.float32),
        mesh=mesh,
        scratch_types=[
            pltpu.VMEM((KCH, 128), jnp.int32),
            pltpu.VMEM((KCH, 128), jnp.int32),
            pltpu.VMEM((KCH, 128), jnp.int32),
            pltpu.VMEM((KCH, 128), jnp.int32),
            pltpu.VMEM((KCH, 128), jnp.float32),
            pltpu.VMEM((KCH, 128), jnp.float32),
            pltpu.VMEM((KCH, 128), jnp.float32),
            pltpu.VMEM((4, NL), jnp.float32),
            pltpu.VMEM((KCH, 128), jnp.float32),
            pltpu.SemaphoreType.DMA,
        ],
        name="meta_rule_sc",
    )
    return f(x_tiles, scal, matflat, p0, p1, p2)


def kernel(x, mat, pred0, pred1, pred2, and_w, and_beta):
    x_tiles = x.astype(jnp.int32).reshape(NW, KCH, 128)
    matflat = mat.astype(jnp.int32).T.reshape(-1)
    p0 = pred0.reshape(1, -1)
    p1 = pred1.reshape(1, -1)
    p2 = pred2.reshape(1, -1)
    scal = jnp.concatenate(
        [jnp.broadcast_to(and_w[i], (1, NL)) for i in range(FORMULA_LEN)]
        + [jnp.broadcast_to(and_beta[0], (1, NL))], axis=0).astype(jnp.float32)
    out = _run(x_tiles, scal, matflat, p0, p1, p2)
    ret = out.reshape(B, 1)
    slacks = jnp.zeros((), dtype=jnp.float32)
    return (ret, slacks)


# speedup vs baseline: 50.6819x; 1.0011x over previous
import jax
import jax.numpy as jnp
from jax import lax
from jax.experimental import pallas as pl
from jax.experimental.pallas import tpu as pltpu
from jax.experimental.pallas import tpu_sc as plsc

NC = 2
NS = 16
NL = 16
NW = NC * NS

B = 16384
FORMULA_LEN = 3
N_PRED = 1000000
BPW = B // NW
KCH = 1
CW = BPW


def _body(x_hbm, scal_hbm, mat_hbm, p0_hbm, p1_hbm, p2_hbm,
          out_hbm, x_v, c0_v, c1_v, c2_v, a0_v, a1_v, a2_v,
          scal_v, out_v, sem):
    wid = lax.axis_index("s") * NC + lax.axis_index("c")

    pltpu.sync_copy(x_hbm.at[wid], x_v)
    pltpu.sync_copy(scal_hbm, scal_v)

    copies = []
    for i, col_v in ((0, c0_v), (1, c1_v), (2, c2_v)):
        view = mat_hbm.at[pl.ds(i * N_PRED, N_PRED)]
        for j in range(KCH):
            copies.append(
                pltpu.make_async_copy(view.at[x_v.at[j]], col_v.at[j], sem))
    for c in copies:
        c.start()
    for c in copies:
        c.wait()

    copies = []
    for p_hbm, col_v, act_v in ((p0_hbm, c0_v, a0_v),
                                (p1_hbm, c1_v, a1_v),
                                (p2_hbm, c2_v, a2_v)):
        for j in range(KCH):
            copies.append(
                pltpu.make_async_copy(p_hbm.at[0].at[col_v.at[j]],
                                      act_v.at[j], sem))
    for c in copies:
        c.start()
    for c in copies:
        c.wait()

    w0 = scal_v[0, :]
    w1 = scal_v[1, :]
    w2 = scal_v[2, :]
    beta = scal_v[3, :]
    one = jnp.ones((NL,), jnp.float32)
    zero = jnp.zeros((NL,), jnp.float32)
    for j in range(KCH):
        for t in range(CW // NL):
            sl = pl.ds(t * NL, NL)
            a0 = a0_v[j, sl]
            a1 = a1_v[j, sl]
            a2 = a2_v[j, sl]
            s = w0 * (one - a0) + w1 * (one - a1) + w2 * (one - a2)
            out_v[j, sl] = jnp.minimum(jnp.maximum(beta - s, zero), one)

    pltpu.sync_copy(out_v, out_hbm.at[wid])


@jax.jit
def _run(x_tiles, scal, matflat, p0, p1, p2):
    mesh = plsc.VectorSubcoreMesh(core_axis_name="c", subcore_axis_name="s",
                                  num_cores=NC, num_subcores=NS)
    f = pl.kernel(
        _body,
        out_type=jax.ShapeDtypeStruct((NW, KCH, CW), jnp.float32),
        mesh=mesh,
        scratch_types=[
            pltpu.VMEM((KCH, CW), jnp.int32),
            pltpu.VMEM((KCH, CW), jnp.int32),
            pltpu.VMEM((KCH, CW), jnp.int32),
            pltpu.VMEM((KCH, CW), jnp.int32),
            pltpu.VMEM((KCH, CW), jnp.float32),
            pltpu.VMEM((KCH, CW), jnp.float32),
            pltpu.VMEM((KCH, CW), jnp.float32),
            pltpu.VMEM((4, NL), jnp.float32),
            pltpu.VMEM((KCH, CW), jnp.float32),
            pltpu.SemaphoreType.DMA,
        ],
        name="meta_rule_sc",
    )
    return f(x_tiles, scal, matflat, p0, p1, p2)


def kernel(x, mat, pred0, pred1, pred2, and_w, and_beta):
    x_tiles = x.astype(jnp.int32).reshape(NW, KCH, CW)
    matflat = mat.astype(jnp.int32).T.reshape(-1)
    p0 = pred0.reshape(1, -1)
    p1 = pred1.reshape(1, -1)
    p2 = pred2.reshape(1, -1)
    scal = jnp.concatenate(
        [jnp.broadcast_to(and_w[i], (1, NL)) for i in range(FORMULA_LEN)]
        + [jnp.broadcast_to(and_beta[0], (1, NL))], axis=0).astype(jnp.float32)
    out = _run(x_tiles, scal, matflat, p0, p1, p2)
    ret = out.reshape(B, 1)
    slacks = jnp.zeros((), dtype=jnp.float32)
    return (ret, slacks)


# speedup vs baseline: 51.4655x; 1.0155x over previous
import jax
import jax.numpy as jnp
from jax import lax
from jax.experimental import pallas as pl
from jax.experimental.pallas import tpu as pltpu
from jax.experimental.pallas import tpu_sc as plsc

NC = 2
NS = 16
NL = 16
NW = NC * NS

B = 16384
FORMULA_LEN = 3
N_PRED = 1000000
BPW = B // NW
KCH = 4
CW = BPW // KCH


def _body(x_hbm, scal_hbm, mat_hbm, p0_hbm, p1_hbm, p2_hbm,
          out_hbm, x_v, c0_v, c1_v, c2_v, a0_v, a1_v, a2_v,
          scal_v, out_v, sems):
    wid = lax.axis_index("s") * NC + lax.axis_index("c")

    pltpu.sync_copy(x_hbm.at[wid], x_v)
    pltpu.sync_copy(scal_hbm, scal_v)

    views = [mat_hbm.at[pl.ds(i * N_PRED, N_PRED)] for i in range(FORMULA_LEN)]
    cols = (c0_v, c1_v, c2_v)
    acts = (a0_v, a1_v, a2_v)
    preds = (p0_hbm, p1_hbm, p2_hbm)

    l1 = [[pltpu.make_async_copy(views[i].at[x_v.at[j]], cols[i].at[j],
                                 sems.at[j])
           for i in range(FORMULA_LEN)] for j in range(KCH)]
    for j in range(KCH):
        for c in l1[j]:
            c.start()
    l2 = []
    for j in range(KCH):
        for c in l1[j]:
            c.wait()
        l2.append([pltpu.make_async_copy(preds[i].at[0].at[cols[i].at[j]],
                                         acts[i].at[j], sems.at[j])
                   for i in range(FORMULA_LEN)])
        for c in l2[j]:
            c.start()

    w0 = scal_v[0, :]
    w1 = scal_v[1, :]
    w2 = scal_v[2, :]
    beta = scal_v[3, :]
    one = jnp.ones((NL,), jnp.float32)
    zero = jnp.zeros((NL,), jnp.float32)
    for j in range(KCH):
        for c in l2[j]:
            c.wait()
        for t in range(CW // NL):
            sl = pl.ds(t * NL, NL)
            a0 = a0_v[j, sl]
            a1 = a1_v[j, sl]
            a2 = a2_v[j, sl]
            s = w0 * (one - a0) + w1 * (one - a1) + w2 * (one - a2)
            out_v[j, sl] = jnp.minimum(jnp.maximum(beta - s, zero), one)

    pltpu.sync_copy(out_v, out_hbm.at[wid])


@jax.jit
def _run(x_tiles, scal, matflat, p0, p1, p2):
    mesh = plsc.VectorSubcoreMesh(core_axis_name="c", subcore_axis_name="s",
                                  num_cores=NC, num_subcores=NS)
    f = pl.kernel(
        _body,
        out_type=jax.ShapeDtypeStruct((NW, KCH, CW), jnp.float32),
        mesh=mesh,
        scratch_types=[
            pltpu.VMEM((KCH, CW), jnp.int32),
            pltpu.VMEM((KCH, CW), jnp.int32),
            pltpu.VMEM((KCH, CW), jnp.int32),
            pltpu.VMEM((KCH, CW), jnp.int32),
            pltpu.VMEM((KCH, CW), jnp.float32),
            pltpu.VMEM((KCH, CW), jnp.float32),
            pltpu.VMEM((KCH, CW), jnp.float32),
            pltpu.VMEM((4, NL), jnp.float32),
            pltpu.VMEM((KCH, CW), jnp.float32),
            pltpu.SemaphoreType.DMA((KCH,)),
        ],
        name="meta_rule_sc",
    )
    return f(x_tiles, scal, matflat, p0, p1, p2)


def kernel(x, mat, pred0, pred1, pred2, and_w, and_beta):
    x_tiles = x.astype(jnp.int32).reshape(NW, KCH, CW)
    matflat = mat.astype(jnp.int32).T.reshape(-1)
    p0 = pred0.reshape(1, -1)
    p1 = pred1.reshape(1, -1)
    p2 = pred2.reshape(1, -1)
    scal = jnp.concatenate(
        [jnp.broadcast_to(and_w[i], (1, NL)) for i in range(FORMULA_LEN)]
        + [jnp.broadcast_to(and_beta[0], (1, NL))], axis=0).astype(jnp.float32)
    out = _run(x_tiles, scal, matflat, p0, p1, p2)
    ret = out.reshape(B, 1)
    slacks = jnp.zeros((), dtype=jnp.float32)
    return (ret, slacks)


# speedup vs baseline: 81.8401x; 1.5902x over previous
import jax
import jax.numpy as jnp
from jax import lax
from jax.experimental import pallas as pl
from jax.experimental.pallas import tpu as pltpu
from jax.experimental.pallas import tpu_sc as plsc

NC = 2
NS = 16
NL = 16
NW = NC * NS

B = 16384
FORMULA_LEN = 3
N_PRED = 1000000
BPW = B // NW
KCH = 4
CW = BPW // KCH


def _depad_body(m_ref, c0_ref, c1_ref, c2_ref):
    c0_ref[...] = m_ref[0, :]
    c1_ref[...] = m_ref[1, :]
    c2_ref[...] = m_ref[2, :]


def _body(x_hbm, scal_hbm, m0_hbm, m1_hbm, m2_hbm, p0_hbm, p1_hbm, p2_hbm,
          out_hbm, x_v, c0_v, c1_v, c2_v, a0_v, a1_v, a2_v,
          scal_v, out_v, sems):
    wid = lax.axis_index("s") * NC + lax.axis_index("c")

    pltpu.sync_copy(x_hbm.at[wid], x_v)
    pltpu.sync_copy(scal_hbm, scal_v)

    views = (m0_hbm, m1_hbm, m2_hbm)
    cols = (c0_v, c1_v, c2_v)
    acts = (a0_v, a1_v, a2_v)
    preds = (p0_hbm, p1_hbm, p2_hbm)

    l1 = [[pltpu.make_async_copy(views[i].at[x_v.at[j]], cols[i].at[j],
                                 sems.at[j])
           for i in range(FORMULA_LEN)] for j in range(KCH)]
    for j in range(KCH):
        for c in l1[j]:
            c.start()
    l2 = []
    for j in range(KCH):
        for c in l1[j]:
            c.wait()
        l2.append([pltpu.make_async_copy(preds[i].at[0].at[cols[i].at[j]],
                                         acts[i].at[j], sems.at[j])
                   for i in range(FORMULA_LEN)])
        for c in l2[j]:
            c.start()

    w0 = scal_v[0, :]
    w1 = scal_v[1, :]
    w2 = scal_v[2, :]
    beta = scal_v[3, :]
    one = jnp.ones((NL,), jnp.float32)
    zero = jnp.zeros((NL,), jnp.float32)
    for j in range(KCH):
        for c in l2[j]:
            c.wait()
        for t in range(CW // NL):
            sl = pl.ds(t * NL, NL)
            a0 = a0_v[j, sl]
            a1 = a1_v[j, sl]
            a2 = a2_v[j, sl]
            s = w0 * (one - a0) + w1 * (one - a1) + w2 * (one - a2)
            out_v[j, sl] = jnp.minimum(jnp.maximum(beta - s, zero), one)

    pltpu.sync_copy(out_v, out_hbm.at[wid])


@jax.jit
def _run(x_tiles, scal, m0, m1, m2, p0, p1, p2):
    mesh = plsc.VectorSubcoreMesh(core_axis_name="c", subcore_axis_name="s",
                                  num_cores=NC, num_subcores=NS)
    f = pl.kernel(
        _body,
        out_type=jax.ShapeDtypeStruct((NW, KCH, CW), jnp.float32),
        mesh=mesh,
        scratch_types=[
            pltpu.VMEM((KCH, CW), jnp.int32),
            pltpu.VMEM((KCH, CW), jnp.int32),
            pltpu.VMEM((KCH, CW), jnp.int32),
            pltpu.VMEM((KCH, CW), jnp.int32),
            pltpu.VMEM((KCH, CW), jnp.float32),
            pltpu.VMEM((KCH, CW), jnp.float32),
            pltpu.VMEM((KCH, CW), jnp.float32),
            pltpu.VMEM((4, NL), jnp.float32),
            pltpu.VMEM((KCH, CW), jnp.float32),
            pltpu.SemaphoreType.DMA((KCH,)),
        ],
        name="meta_rule_sc",
    )
    return f(x_tiles, scal, m0, m1, m2, p0, p1, p2)


def kernel(x, mat, pred0, pred1, pred2, and_w, and_beta):
    x_tiles = x.astype(jnp.int32).reshape(NW, KCH, CW)
    m0, m1, m2 = pl.pallas_call(
        _depad_body,
        grid=(1,),
        in_specs=[pl.BlockSpec((FORMULA_LEN, N_PRED), lambda i: (0, 0))],
        out_specs=[pl.BlockSpec((N_PRED,), lambda i: (0,))] * FORMULA_LEN,
        out_shape=[jax.ShapeDtypeStruct((N_PRED,), jnp.int32)] * FORMULA_LEN,
    )(mat.astype(jnp.int32).T)
    p0 = pred0.reshape(1, -1)
    p1 = pred1.reshape(1, -1)
    p2 = pred2.reshape(1, -1)
    scal = jnp.concatenate(
        [jnp.broadcast_to(and_w[i], (1, NL)) for i in range(FORMULA_LEN)]
        + [jnp.broadcast_to(and_beta[0], (1, NL))], axis=0).astype(jnp.float32)
    out = _run(x_tiles, scal, m0, m1, m2, p0, p1, p2)
    ret = out.reshape(B, 1)
    slacks = jnp.zeros((), dtype=jnp.float32)
    return (ret, slacks)


# speedup vs baseline: 90.7887x; 1.1093x over previous
import jax
import jax.numpy as jnp
from jax import lax
from jax.experimental import pallas as pl
from jax.experimental.pallas import tpu as pltpu
from jax.experimental.pallas import tpu_sc as plsc

NC = 2
NS = 16
NL = 16
NW = NC * NS

B = 16384
FORMULA_LEN = 3
N_PRED = 1000000
BPW = B // NW
KCH = 4
CW = BPW // KCH


def _depad_body(m_ref, w_ref, b_ref, c0_ref, c1_ref, c2_ref, scal_ref):
    c0_ref[...] = m_ref[0, :]
    c1_ref[...] = m_ref[1, :]
    c2_ref[...] = m_ref[2, :]
    wb = jnp.concatenate([w_ref[...], b_ref[...]], axis=0)
    scal_ref[...] = jnp.broadcast_to(wb[:, None], (4, NL))


def _body(x_hbm, scal_hbm, m0_hbm, m1_hbm, m2_hbm, p0_hbm, p1_hbm, p2_hbm,
          out_hbm, x_v, c0_v, c1_v, c2_v, a0_v, a1_v, a2_v,
          scal_v, out_v, sems):
    wid = lax.axis_index("s") * NC + lax.axis_index("c")

    pltpu.sync_copy(x_hbm.at[wid], x_v)
    pltpu.sync_copy(scal_hbm, scal_v)

    views = (m0_hbm, m1_hbm, m2_hbm)
    cols = (c0_v, c1_v, c2_v)
    acts = (a0_v, a1_v, a2_v)
    preds = (p0_hbm, p1_hbm, p2_hbm)

    l1 = [[pltpu.make_async_copy(views[i].at[x_v.at[j]], cols[i].at[j],
                                 sems.at[j])
           for i in range(FORMULA_LEN)] for j in range(KCH)]
    for j in range(KCH):
        for c in l1[j]:
            c.start()
    l2 = []
    for j in range(KCH):
        for c in l1[j]:
            c.wait()
        l2.append([pltpu.make_async_copy(preds[i].at[0].at[cols[i].at[j]],
                                         acts[i].at[j], sems.at[j])
                   for i in range(FORMULA_LEN)])
        for c in l2[j]:
            c.start()

    w0 = scal_v[0, :]
    w1 = scal_v[1, :]
    w2 = scal_v[2, :]
    beta = scal_v[3, :]
    one = jnp.ones((NL,), jnp.float32)
    zero = jnp.zeros((NL,), jnp.float32)
    for j in range(KCH):
        for c in l2[j]:
            c.wait()
        for t in range(CW // NL):
            sl = pl.ds(t * NL, NL)
            a0 = a0_v[j, sl]
            a1 = a1_v[j, sl]
            a2 = a2_v[j, sl]
            s = w0 * (one - a0) + w1 * (one - a1) + w2 * (one - a2)
            out_v[j, sl] = jnp.minimum(jnp.maximum(beta - s, zero), one)

    pltpu.sync_copy(out_v, out_hbm.at[wid])


@jax.jit
def _run(x_tiles, scal, m0, m1, m2, p0, p1, p2):
    mesh = plsc.VectorSubcoreMesh(core_axis_name="c", subcore_axis_name="s",
                                  num_cores=NC, num_subcores=NS)
    f = pl.kernel(
        _body,
        out_type=jax.ShapeDtypeStruct((NW, KCH, CW), jnp.float32),
        mesh=mesh,
        scratch_types=[
            pltpu.VMEM((KCH, CW), jnp.int32),
            pltpu.VMEM((KCH, CW), jnp.int32),
            pltpu.VMEM((KCH, CW), jnp.int32),
            pltpu.VMEM((KCH, CW), jnp.int32),
            pltpu.VMEM((KCH, CW), jnp.float32),
            pltpu.VMEM((KCH, CW), jnp.float32),
            pltpu.VMEM((KCH, CW), jnp.float32),
            pltpu.VMEM((4, NL), jnp.float32),
            pltpu.VMEM((KCH, CW), jnp.float32),
            pltpu.SemaphoreType.DMA((KCH,)),
        ],
        name="meta_rule_sc",
    )
    return f(x_tiles, scal, m0, m1, m2, p0, p1, p2)


def kernel(x, mat, pred0, pred1, pred2, and_w, and_beta):
    x_tiles = x.astype(jnp.int32).reshape(NW, KCH, CW)
    m0, m1, m2, scal = pl.pallas_call(
        _depad_body,
        grid=(1,),
        in_specs=[pl.BlockSpec((FORMULA_LEN, N_PRED), lambda i: (0, 0)),
                  pl.BlockSpec((FORMULA_LEN,), lambda i: (0,)),
                  pl.BlockSpec((1,), lambda i: (0,))],
        out_specs=[pl.BlockSpec((N_PRED,), lambda i: (0,))] * FORMULA_LEN
        + [pl.BlockSpec((4, NL), lambda i: (0, 0))],
        out_shape=[jax.ShapeDtypeStruct((N_PRED,), jnp.int32)] * FORMULA_LEN
        + [jax.ShapeDtypeStruct((4, NL), jnp.float32)],
    )(mat.astype(jnp.int32).T, and_w.astype(jnp.float32),
      and_beta.astype(jnp.float32))
    p0 = pred0.reshape(1, -1)
    p1 = pred1.reshape(1, -1)
    p2 = pred2.reshape(1, -1)
    out = _run(x_tiles, scal, m0, m1, m2, p0, p1, p2)
    ret = out.reshape(B, 1)
    slacks = jnp.zeros((), dtype=jnp.float32)
    return (ret, slacks)
